# baseline (speedup 1.0000x reference)
import functools

import jax
import jax.numpy as jnp
from jax.experimental import pallas as pl
from jax.experimental.pallas import tpu as pltpu

_K = 20
_CHUNK = 256


def _edge_kernel(q_ref, kt_ref, ksq_ref, out_ref, *, k, chunk):
    n = kt_ref.shape[2]
    kt = kt_ref[0]
    key_sq = ksq_ref[0]

    idx_bits = max(1, (n - 1).bit_length())
    low_mask = (1 << idx_bits) - 1
    high_mask = jnp.int32(~low_mask)
    lane = jax.lax.broadcasted_iota(jnp.int32, (1, n), 1)

    out_ref[1, 0] = jax.lax.broadcasted_iota(jnp.int32, (n, k), 0)

    col = jax.lax.broadcasted_iota(jnp.int32, (chunk, k), 1)

    for c in range(n // chunk):
        q = q_ref[0, c * chunk:(c + 1) * chunk, :]
        inner = jnp.dot(q, kt, preferred_element_type=jnp.float32)
        rank = key_sq - 2.0 * inner

        cur = pltpu.bitcast(
            (pltpu.bitcast(rank, jnp.int32) & high_mask) | lane,
            jnp.float32)

        prev = jnp.full((chunk, 1), -jnp.inf, dtype=jnp.float32)
        acc = jnp.zeros((chunk, k), dtype=jnp.int32)
        for j in range(k):
            cand = jnp.where(cur > prev, cur, jnp.inf)
            sel = jnp.min(cand, axis=-1, keepdims=True)
            sel_idx = pltpu.bitcast(sel, jnp.int32) & low_mask
            acc = jnp.where(col == j, sel_idx, acc)
            prev = sel
        out_ref[0, 0, c * chunk:(c + 1) * chunk, :] = acc


def kernel(x):
    B, C, N, _ = x.shape
    k = _K

    xp = jnp.transpose(jnp.squeeze(x, -1), (0, 2, 1)).astype(jnp.float32)
    col_norm = jnp.sqrt(jnp.sum(xp * xp, axis=1, keepdims=True))
    xn = xp / jnp.maximum(col_norm, 1e-12)
    key_sq = jnp.transpose(
        jnp.sum(xn * xn, axis=-1, keepdims=True), (0, 2, 1))
    xnT = jnp.transpose(xn, (0, 2, 1))

    edge = pl.pallas_call(
        functools.partial(_edge_kernel, k=k, chunk=_CHUNK),
        out_shape=jax.ShapeDtypeStruct((2, B, N, k), jnp.int32),
        grid=(B,),
        in_specs=[
            pl.BlockSpec((1, N, C), lambda b: (b, 0, 0)),
            pl.BlockSpec((1, C, N), lambda b: (b, 0, 0)),
            pl.BlockSpec((1, 1, N), lambda b: (b, 0, 0)),
        ],
        out_specs=pl.BlockSpec((2, 1, N, k), lambda b: (0, b, 0, 0)),
        compiler_params=pltpu.CompilerParams(
            dimension_semantics=("parallel",),
            vmem_limit_bytes=48 << 20),
    )(xn, xnT, key_sq)
    return edge
